# baseline (speedup 1.0000x reference)
import functools

import jax
import jax.numpy as jnp
from jax import lax
from jax.experimental import pallas as pl
from jax.experimental.pallas import tpu as pltpu
from jax.experimental.pallas import tpu_sc as plsc

N = 2048
FIN = 128
FOUT = 32
NC = 2
NS = 16
NW = NC * NS
SC_ROWS = 1024
TC_ROWS = N - SC_ROWS
ROWS_PER = SC_ROWS // NW
RBLK = 2
NGRP = N // 16
BLK = 128


def _prep_body(x_ref, w_ref, a_ref, h_ref, s_ref, t_ref):
    h = jnp.dot(x_ref[...], w_ref[...], preferred_element_type=jnp.float32)
    h_ref[...] = h
    a0 = a_ref[0, :FOUT]
    a1 = a_ref[0, FOUT:]
    s_ref[...] = jnp.dot(h, a0, preferred_element_type=jnp.float32)[None, :]
    t_ref[...] = jnp.dot(h, a1, preferred_element_type=jnp.float32)[None, :]


def _tc_prep(x, W, a):
    return pl.pallas_call(
        _prep_body,
        out_shape=(
            jax.ShapeDtypeStruct((N, FOUT), jnp.float32),
            jax.ShapeDtypeStruct((1, N), jnp.float32),
            jax.ShapeDtypeStruct((1, N), jnp.float32),
        ),
    )(x, W, a)


def _tc_attn_body(adj_ref, h_ref, s_ref, t_ref, out_ref):
    i = pl.program_id(0)
    h = h_ref[...]
    s_blk = s_ref[0, pl.ds(SC_ROWS + i * BLK, BLK)]
    u = s_blk[:, None] + t_ref[0, :][None, :]
    lr = jnp.maximum(u, 0.2 * u)
    e = jnp.exp(-jnp.clip(lr, -50.0, 50.0))
    w = jnp.where(adj_ref[...] != 0, e, 0.0)
    numer = jnp.dot(w, h, preferred_element_type=jnp.float32)
    denom = jnp.sum(w, axis=1)
    hp = numer / denom[:, None]
    out_ref[...] = jnp.where(hp > 0, hp, jnp.exp(hp) - 1.0)


def _tc_attn(adj, h, s2, t2):
    return pl.pallas_call(
        _tc_attn_body,
        grid=(TC_ROWS // BLK,),
        in_specs=[
            pl.BlockSpec((BLK, N), lambda i: (i + SC_ROWS // BLK, 0)),
            pl.BlockSpec((N, FOUT), lambda i: (0, 0)),
            pl.BlockSpec((1, N), lambda i: (0, 0)),
            pl.BlockSpec((1, N), lambda i: (0, 0)),
        ],
        out_specs=pl.BlockSpec((BLK, FOUT), lambda i: (i, 0)),
        out_shape=jax.ShapeDtypeStruct((TC_ROWS, FOUT), jnp.float32),
        compiler_params=pltpu.CompilerParams(
            dimension_semantics=("arbitrary",),
        ),
    )(adj, h, s2, t2)


def _sc_body(adj_hbm, h_hbm, s_hbm, t_hbm, out_hbm,
             h_v, t_v, s_v, a_buf, w_buf, out_v):
    wid = lax.axis_index("s") * NC + lax.axis_index("c")
    row0 = wid * ROWS_PER
    pltpu.sync_copy(h_hbm, h_v)
    pltpu.sync_copy(t_hbm, t_v)
    pltpu.sync_copy(s_hbm.at[pl.ds(row0, ROWS_PER)], s_v)

    def super_body(sc, carry_outer):
        i0 = sc * 16
        s16 = s_v[pl.ds(i0, 16)]
        for cc in range(16 // RBLK):
            pltpu.sync_copy(
                adj_hbm.at[pl.ds(row0 + i0 + cc * RBLK, RBLK), :], a_buf)

            def w_body(g, carry, _cc=cc):
                rs = list(carry)
                j0 = g * 16
                t_vec = t_v[pl.ds(j0, 16)]
                for r in range(RBLK):
                    a_vec = a_buf[r, pl.ds(j0, 16)]
                    u = s16[_cc * RBLK + r] + t_vec
                    lr = jnp.maximum(u, 0.2 * u)
                    e = jnp.exp(-jnp.clip(lr, -50.0, 50.0))
                    w = jnp.where(a_vec != 0, e, 0.0)
                    w_buf[pl.ds(r * N + j0, 16)] = w
                    rs[r] = rs[r] + w
                return tuple(rs)

            zero = jnp.zeros((16,), jnp.float32)
            rs = plsc.parallel_loop(
                0, NGRP, carry=tuple(zero for _ in range(RBLK)))(
                    lambda g, c: w_body(g, c))

            def g_body(g, carry):
                lo = list(carry[0])
                hi = list(carry[1])
                j0 = g * 16
                ws = [w_buf[pl.ds(r * N + j0, 16)] for r in range(RBLK)]
                for l in range(16):
                    hlo = h_v[pl.ds((j0 + l) * FOUT, 16)]
                    hhi = h_v[pl.ds((j0 + l) * FOUT + 16, 16)]
                    for r in range(RBLK):
                        wl = ws[r][l]
                        lo[r] = lo[r] + wl * hlo
                        hi[r] = hi[r] + wl * hhi
                return (tuple(lo), tuple(hi))

            init = (tuple(zero for _ in range(RBLK)),
                    tuple(zero for _ in range(RBLK)))
            lo, hi = plsc.parallel_loop(0, NGRP, unroll=2, carry=init)(
                lambda g, c: g_body(g, c))
            for r in range(RBLK):
                dv = rs[r]
                denom = dv[0]
                for l in range(1, 16):
                    denom = denom + dv[l]
                plo = lo[r] / denom
                phi = hi[r] / denom
                plo = jnp.where(plo > 0, plo, jnp.exp(plo) - 1.0)
                phi = jnp.where(phi > 0, phi, jnp.exp(phi) - 1.0)
                ob = (i0 + cc * RBLK + r) * FOUT
                out_v[pl.ds(ob, 16)] = plo
                out_v[pl.ds(ob + 16, 16)] = phi
        return carry_outer

    lax.fori_loop(0, ROWS_PER // 16, super_body, 0)
    pltpu.sync_copy(out_v, out_hbm.at[pl.ds(row0 * FOUT, ROWS_PER * FOUT)])


@functools.partial(
    pl.kernel,
    mesh=plsc.VectorSubcoreMesh(core_axis_name="c", subcore_axis_name="s"),
    out_type=jax.ShapeDtypeStruct((SC_ROWS * FOUT,), jnp.float32),
    scratch_types=[
        pltpu.VMEM((N * FOUT,), jnp.float32),
        pltpu.VMEM((N,), jnp.float32),
        pltpu.VMEM((ROWS_PER,), jnp.float32),
        pltpu.VMEM((RBLK, N), jnp.int32),
        pltpu.VMEM((RBLK * N,), jnp.float32),
        pltpu.VMEM((ROWS_PER * FOUT,), jnp.float32),
    ],
)
def _sc_aggregate(adj_hbm, h_hbm, s_hbm, t_hbm, out_hbm,
                  h_v, t_v, s_v, a_buf, w_buf, out_v):
    _sc_body(adj_hbm, h_hbm, s_hbm, t_hbm, out_hbm,
             h_v, t_v, s_v, a_buf, w_buf, out_v)


@jax.jit
def kernel(x, adj, W, a):
    h, s2, t2 = _tc_prep(x, W, a)
    s = s2.reshape(N)
    t = t2.reshape(N)
    out_tc = _tc_attn(adj, h, s2, t2)
    out_sc = _sc_aggregate(adj, h.reshape(N * FOUT), s, t)
    return jnp.concatenate(
        [out_sc.reshape(SC_ROWS, FOUT), out_tc], axis=0)


# speedup vs baseline: 1.2837x; 1.2837x over previous
import functools

import jax
import jax.numpy as jnp
from jax import lax
from jax.experimental import pallas as pl
from jax.experimental.pallas import tpu as pltpu
from jax.experimental.pallas import tpu_sc as plsc

N = 2048
FIN = 128
FOUT = 32
NC = 2
NS = 16
NW = NC * NS
SC_ROWS = 1024
TC_ROWS = N - SC_ROWS
ROWS_PER = SC_ROWS // NW
RBLK = 2
NGRP = N // 16
BLK = 128


def _prep_body(x_ref, w_ref, a_ref, h_ref, s_ref, t_ref):
    h = jnp.dot(x_ref[...], w_ref[...], preferred_element_type=jnp.float32)
    h_ref[...] = h
    a0 = a_ref[0, :FOUT]
    a1 = a_ref[0, FOUT:]
    s_ref[...] = jnp.dot(h, a0, preferred_element_type=jnp.float32)[None, :]
    t_ref[...] = jnp.dot(h, a1, preferred_element_type=jnp.float32)[None, :]


def _tc_prep(x, W, a):
    return pl.pallas_call(
        _prep_body,
        out_shape=(
            jax.ShapeDtypeStruct((N, FOUT), jnp.float32),
            jax.ShapeDtypeStruct((1, N), jnp.float32),
            jax.ShapeDtypeStruct((1, N), jnp.float32),
        ),
    )(x, W, a)


def _tc_attn_body(adj_ref, h_ref, s_ref, t_ref, out_ref):
    i = pl.program_id(0)
    h = h_ref[...]
    s_blk = s_ref[0, pl.ds(SC_ROWS + i * BLK, BLK)]
    u = s_blk[:, None] + t_ref[0, :][None, :]
    lr = jnp.maximum(u, 0.2 * u)
    e = jnp.exp(-jnp.clip(lr, -50.0, 50.0))
    w = jnp.where(adj_ref[...] != 0, e, 0.0)
    numer = jnp.dot(w, h, preferred_element_type=jnp.float32)
    denom = jnp.sum(w, axis=1)
    hp = numer / denom[:, None]
    out_ref[...] = jnp.where(hp > 0, hp, jnp.exp(hp) - 1.0)


def _tc_attn(adj, h, s2, t2):
    return pl.pallas_call(
        _tc_attn_body,
        grid=(TC_ROWS // BLK,),
        in_specs=[
            pl.BlockSpec((BLK, N), lambda i: (i + SC_ROWS // BLK, 0)),
            pl.BlockSpec((N, FOUT), lambda i: (0, 0)),
            pl.BlockSpec((1, N), lambda i: (0, 0)),
            pl.BlockSpec((1, N), lambda i: (0, 0)),
        ],
        out_specs=pl.BlockSpec((BLK, FOUT), lambda i: (i, 0)),
        out_shape=jax.ShapeDtypeStruct((TC_ROWS, FOUT), jnp.float32),
        compiler_params=pltpu.CompilerParams(
            dimension_semantics=("arbitrary",),
        ),
    )(adj, h, s2, t2)


def _tc_wgen_body(adj_ref, s_ref, t_ref, wn_ref):
    i = pl.program_id(0)
    s_blk = s_ref[0, pl.ds(i * BLK, BLK)]
    u = s_blk[:, None] + t_ref[0, :][None, :]
    lr = jnp.maximum(u, 0.2 * u)
    e = jnp.exp(-jnp.clip(lr, -50.0, 50.0))
    w = jnp.where(adj_ref[...] != 0, e, 0.0)
    dn = jnp.sum(w, axis=1)
    wn_ref[...] = w * (1.0 / dn)[:, None]


def _tc_wgen(adj, s2, t2):
    return pl.pallas_call(
        _tc_wgen_body,
        grid=(SC_ROWS // BLK,),
        in_specs=[
            pl.BlockSpec((BLK, N), lambda i: (i, 0)),
            pl.BlockSpec((1, N), lambda i: (0, 0)),
            pl.BlockSpec((1, N), lambda i: (0, 0)),
        ],
        out_specs=pl.BlockSpec((BLK, N), lambda i: (i, 0)),
        out_shape=jax.ShapeDtypeStruct((SC_ROWS, N), jnp.float32),
        compiler_params=pltpu.CompilerParams(
            dimension_semantics=("arbitrary",),
        ),
    )(adj, s2, t2)


def _sc_body(wn_hbm, h_hbm, out_hbm, h_v, wv_buf, out_v):
    wid = lax.axis_index("s") * NC + lax.axis_index("c")
    row0 = wid * ROWS_PER
    pltpu.sync_copy(h_hbm, h_v)

    def chunk_body(c, carry_outer):
        pltpu.sync_copy(
            wn_hbm.at[pl.ds(row0 + c * RBLK, RBLK), :], wv_buf)

        def g_body(g, carry):
            lo = list(carry[0])
            hi = list(carry[1])
            j0 = g * 16
            ws = [wv_buf[r, pl.ds(j0, 16)] for r in range(RBLK)]
            for l in range(16):
                hlo = h_v[pl.ds((j0 + l) * FOUT, 16)]
                hhi = h_v[pl.ds((j0 + l) * FOUT + 16, 16)]
                for r in range(RBLK):
                    wl = ws[r][l]
                    lo[r] = lo[r] + wl * hlo
                    hi[r] = hi[r] + wl * hhi
            return (tuple(lo), tuple(hi))

        zero = jnp.zeros((16,), jnp.float32)
        init = (tuple(zero for _ in range(RBLK)),
                tuple(zero for _ in range(RBLK)))
        lo, hi = plsc.parallel_loop(0, NGRP, unroll=2, carry=init)(
            lambda g, c: g_body(g, c))
        for r in range(RBLK):
            plo = lo[r]
            phi = hi[r]
            plo = jnp.where(plo > 0, plo, jnp.exp(plo) - 1.0)
            phi = jnp.where(phi > 0, phi, jnp.exp(phi) - 1.0)
            ob = (c * RBLK + r) * FOUT
            out_v[pl.ds(ob, 16)] = plo
            out_v[pl.ds(ob + 16, 16)] = phi
        return carry_outer

    lax.fori_loop(0, ROWS_PER // RBLK, chunk_body, 0)
    pltpu.sync_copy(out_v, out_hbm.at[pl.ds(row0 * FOUT, ROWS_PER * FOUT)])


@functools.partial(
    pl.kernel,
    mesh=plsc.VectorSubcoreMesh(core_axis_name="c", subcore_axis_name="s"),
    out_type=jax.ShapeDtypeStruct((SC_ROWS * FOUT,), jnp.float32),
    scratch_types=[
        pltpu.VMEM((N * FOUT,), jnp.float32),
        pltpu.VMEM((RBLK, N), jnp.float32),
        pltpu.VMEM((ROWS_PER * FOUT,), jnp.float32),
    ],
)
def _sc_aggregate(wn_hbm, h_hbm, out_hbm, h_v, wv_buf, out_v):
    _sc_body(wn_hbm, h_hbm, out_hbm, h_v, wv_buf, out_v)


@jax.jit
def kernel(x, adj, W, a):
    h, s2, t2 = _tc_prep(x, W, a)
    wn = _tc_wgen(adj, s2, t2)
    out_sc = _sc_aggregate(wn, h.reshape(N * FOUT))
    out_tc = _tc_attn(adj, h, s2, t2)
    return jnp.concatenate(
        [out_sc.reshape(SC_ROWS, FOUT), out_tc], axis=0)


# speedup vs baseline: 1.5601x; 1.2153x over previous
import functools

import jax
import jax.numpy as jnp
from jax import lax
from jax.experimental import pallas as pl
from jax.experimental.pallas import tpu as pltpu
from jax.experimental.pallas import tpu_sc as plsc

N = 2048
FIN = 128
FOUT = 32
NC = 2
NS = 16
NW = NC * NS
SC_ROWS = 768
TC_ROWS = N - SC_ROWS
ROWS_PER = SC_ROWS // NW
RBLK = 2
NGRP = N // 16
BLK = 128


def _prep_body(x_ref, w_ref, a_ref, h_ref, s_ref, t_ref):
    h = jnp.dot(x_ref[...], w_ref[...], preferred_element_type=jnp.float32)
    h_ref[...] = h
    a0 = a_ref[0, :FOUT]
    a1 = a_ref[0, FOUT:]
    s_ref[...] = jnp.dot(h, a0, preferred_element_type=jnp.float32)[None, :]
    t_ref[...] = jnp.dot(h, a1, preferred_element_type=jnp.float32)[None, :]


def _tc_prep(x, W, a):
    return pl.pallas_call(
        _prep_body,
        out_shape=(
            jax.ShapeDtypeStruct((N, FOUT), jnp.float32),
            jax.ShapeDtypeStruct((1, N), jnp.float32),
            jax.ShapeDtypeStruct((1, N), jnp.float32),
        ),
    )(x, W, a)


def _tc_attn_body(adj_ref, h_ref, s_ref, t_ref, out_ref):
    i = pl.program_id(0)
    h = h_ref[...]
    s_blk = s_ref[0, pl.ds(SC_ROWS + i * BLK, BLK)]
    u = s_blk[:, None] + t_ref[0, :][None, :]
    lr = jnp.maximum(u, 0.2 * u)
    e = jnp.exp(-jnp.clip(lr, -50.0, 50.0))
    w = jnp.where(adj_ref[...] != 0, e, 0.0)
    numer = jnp.dot(w, h, preferred_element_type=jnp.float32)
    denom = jnp.sum(w, axis=1)
    hp = numer / denom[:, None]
    out_ref[...] = jnp.where(hp > 0, hp, jnp.exp(hp) - 1.0)


def _tc_attn(adj, h, s2, t2):
    return pl.pallas_call(
        _tc_attn_body,
        grid=(TC_ROWS // BLK,),
        in_specs=[
            pl.BlockSpec((BLK, N), lambda i: (i + SC_ROWS // BLK, 0)),
            pl.BlockSpec((N, FOUT), lambda i: (0, 0)),
            pl.BlockSpec((1, N), lambda i: (0, 0)),
            pl.BlockSpec((1, N), lambda i: (0, 0)),
        ],
        out_specs=pl.BlockSpec((BLK, FOUT), lambda i: (i, 0)),
        out_shape=jax.ShapeDtypeStruct((TC_ROWS, FOUT), jnp.float32),
        compiler_params=pltpu.CompilerParams(
            dimension_semantics=("arbitrary",),
        ),
    )(adj, h, s2, t2)


def _tc_wgen_body(adj_ref, s_ref, t_ref, wn_ref):
    i = pl.program_id(0)
    s_blk = s_ref[0, pl.ds(i * BLK, BLK)]
    u = s_blk[:, None] + t_ref[0, :][None, :]
    lr = jnp.maximum(u, 0.2 * u)
    e = jnp.exp(-jnp.clip(lr, -50.0, 50.0))
    w = jnp.where(adj_ref[...] != 0, e, 0.0)
    dn = jnp.sum(w, axis=1)
    wn_ref[...] = w * (1.0 / dn)[:, None]


def _tc_wgen(adj, s2, t2):
    return pl.pallas_call(
        _tc_wgen_body,
        grid=(SC_ROWS // BLK,),
        in_specs=[
            pl.BlockSpec((BLK, N), lambda i: (i, 0)),
            pl.BlockSpec((1, N), lambda i: (0, 0)),
            pl.BlockSpec((1, N), lambda i: (0, 0)),
        ],
        out_specs=pl.BlockSpec((BLK, N), lambda i: (i, 0)),
        out_shape=jax.ShapeDtypeStruct((SC_ROWS, N), jnp.float32),
        compiler_params=pltpu.CompilerParams(
            dimension_semantics=("arbitrary",),
        ),
    )(adj, s2, t2)


def _sc_body(wn_hbm, h_hbm, out_hbm, h_v, wv_buf, out_v):
    wid = lax.axis_index("s") * NC + lax.axis_index("c")
    row0 = wid * ROWS_PER
    pltpu.sync_copy(h_hbm, h_v)

    def chunk_body(c, carry_outer):
        pltpu.sync_copy(
            wn_hbm.at[pl.ds(row0 + c * RBLK, RBLK), :], wv_buf)

        def g_body(g, carry):
            lo = list(carry[0])
            hi = list(carry[1])
            j0 = g * 16
            ws = [wv_buf[r, pl.ds(j0, 16)] for r in range(RBLK)]
            for l in range(16):
                hlo = h_v[pl.ds((j0 + l) * FOUT, 16)]
                hhi = h_v[pl.ds((j0 + l) * FOUT + 16, 16)]
                for r in range(RBLK):
                    wl = ws[r][l]
                    lo[r] = lo[r] + wl * hlo
                    hi[r] = hi[r] + wl * hhi
            return (tuple(lo), tuple(hi))

        zero = jnp.zeros((16,), jnp.float32)
        init = (tuple(zero for _ in range(RBLK)),
                tuple(zero for _ in range(RBLK)))
        lo, hi = plsc.parallel_loop(0, NGRP, unroll=2, carry=init)(
            lambda g, c: g_body(g, c))
        for r in range(RBLK):
            plo = lo[r]
            phi = hi[r]
            plo = jnp.where(plo > 0, plo, jnp.exp(plo) - 1.0)
            phi = jnp.where(phi > 0, phi, jnp.exp(phi) - 1.0)
            ob = (c * RBLK + r) * FOUT
            out_v[pl.ds(ob, 16)] = plo
            out_v[pl.ds(ob + 16, 16)] = phi
        return carry_outer

    lax.fori_loop(0, ROWS_PER // RBLK, chunk_body, 0)
    pltpu.sync_copy(out_v, out_hbm.at[pl.ds(row0 * FOUT, ROWS_PER * FOUT)])


@functools.partial(
    pl.kernel,
    mesh=plsc.VectorSubcoreMesh(core_axis_name="c", subcore_axis_name="s"),
    out_type=jax.ShapeDtypeStruct((SC_ROWS * FOUT,), jnp.float32),
    scratch_types=[
        pltpu.VMEM((N * FOUT,), jnp.float32),
        pltpu.VMEM((RBLK, N), jnp.float32),
        pltpu.VMEM((ROWS_PER * FOUT,), jnp.float32),
    ],
)
def _sc_aggregate(wn_hbm, h_hbm, out_hbm, h_v, wv_buf, out_v):
    _sc_body(wn_hbm, h_hbm, out_hbm, h_v, wv_buf, out_v)


@jax.jit
def kernel(x, adj, W, a):
    h, s2, t2 = _tc_prep(x, W, a)
    wn = _tc_wgen(adj, s2, t2)
    out_sc = _sc_aggregate(wn, h.reshape(N * FOUT))
    out_tc = _tc_attn(adj, h, s2, t2)
    return jnp.concatenate(
        [out_sc.reshape(SC_ROWS, FOUT), out_tc], axis=0)
